# Initial kernel scaffold; baseline (speedup 1.0000x reference)
import functools

import jax
import jax.numpy as jnp
from jax import lax
from jax.experimental import pallas as pl
from jax.experimental.pallas import tpu as pltpu
from jax.experimental.pallas import tpu_sc as plsc


def _emb_call(B, D, NC, NS):
    NW = NC * NS
    b_per_w = B // NW
    CHUNK = 1600
    n_chunks = b_per_w // CHUNK

    mesh = plsc.VectorSubcoreMesh(core_axis_name="c", subcore_axis_name="s")

    @functools.partial(
        pl.kernel,
        out_type=jax.ShapeDtypeStruct((B, D), jnp.float32),
        mesh=mesh,
        scratch_types=[
            pltpu.VMEM((CHUNK,), jnp.int32),
            pltpu.VMEM((CHUNK, D), jnp.float32),
            pltpu.SemaphoreType.DMA,
        ],
    )
    def emb(x_hbm, table_hbm, out_hbm, idx_v, rows_v, sem):
        wid = lax.axis_index("s") * NC + lax.axis_index("c")
        base0 = wid * b_per_w

        def body(i, carry):
            base = base0 + i * CHUNK
            pltpu.sync_copy(x_hbm.at[pl.ds(base, CHUNK)], idx_v)
            pltpu.async_copy(table_hbm.at[idx_v], rows_v, sem).wait()
            pltpu.sync_copy(rows_v, out_hbm.at[pl.ds(base, CHUNK)])
            return carry

        lax.fori_loop(0, n_chunks, body, 0)

    return emb


def kernel(x, table):
    Bm, Bh = x.shape
    B = Bm * Bh
    D = table.shape[1]
    emb = _emb_call(B, D, 2, 16)
    xf = x.reshape(-1).astype(jnp.int32)
    out = emb(xf, table)
    return out.reshape(Bm, Bh, D)


# speedup vs baseline: 1.1026x; 1.1026x over previous
import functools

import jax
import jax.numpy as jnp
from jax import lax
from jax.experimental import pallas as pl
from jax.experimental.pallas import tpu as pltpu
from jax.experimental.pallas import tpu_sc as plsc


def _emb_call(B, D, NC, NS):
    NW = NC * NS
    b_per_w = B // NW
    CHUNK = 1600
    n_chunks = b_per_w // CHUNK

    mesh = plsc.VectorSubcoreMesh(core_axis_name="c", subcore_axis_name="s")

    @functools.partial(
        pl.kernel,
        out_type=jax.ShapeDtypeStruct((B, D), jnp.float32),
        mesh=mesh,
        scratch_types=[
            pltpu.VMEM((CHUNK,), jnp.int32),
            pltpu.VMEM((CHUNK, D), jnp.float32),
            pltpu.SemaphoreType.DMA,
        ],
        compiler_params=pltpu.CompilerParams(use_tc_tiling_on_sc=False),
    )
    def emb(x_hbm, table_hbm, out_hbm, idx_v, rows_v, sem):
        wid = lax.axis_index("s") * NC + lax.axis_index("c")
        base0 = wid * b_per_w

        def body(i, carry):
            base = base0 + i * CHUNK
            pltpu.sync_copy(x_hbm.at[pl.ds(base, CHUNK)], idx_v)
            pltpu.async_copy(table_hbm.at[idx_v], rows_v, sem).wait()
            pltpu.sync_copy(rows_v, out_hbm.at[pl.ds(base, CHUNK)])
            return carry

        lax.fori_loop(0, n_chunks, body, 0)

    return emb


def kernel(x, table):
    Bm, Bh = x.shape
    B = Bm * Bh
    D = table.shape[1]
    emb = _emb_call(B, D, 2, 16)
    xf = x.reshape(-1).astype(jnp.int32)
    out = emb(xf, table)
    return out.reshape(Bm, Bh, D)


# speedup vs baseline: 1.1134x; 1.0098x over previous
import functools

import jax
import jax.numpy as jnp
from jax import lax
from jax.experimental import pallas as pl
from jax.experimental.pallas import tpu as pltpu
from jax.experimental.pallas import tpu_sc as plsc

_CHUNK = 800
_NBUF = 4


def _emb_call(B, D, NC, NS):
    NW = NC * NS
    b_per_w = B // NW
    n_chunks = b_per_w // _CHUNK
    n_groups = n_chunks // _NBUF

    mesh = plsc.VectorSubcoreMesh(core_axis_name="c", subcore_axis_name="s")

    @functools.partial(
        pl.kernel,
        out_type=jax.ShapeDtypeStruct((B, D), jnp.float32),
        mesh=mesh,
        scratch_types=[
            pltpu.VMEM((b_per_w,), jnp.int32),
            [pltpu.VMEM((_CHUNK, D), jnp.float32) for _ in range(_NBUF)],
            [pltpu.SemaphoreType.DMA for _ in range(_NBUF)],
            [pltpu.SemaphoreType.DMA for _ in range(_NBUF)],
        ],
        compiler_params=pltpu.CompilerParams(use_tc_tiling_on_sc=False),
    )
    def emb(x_hbm, table_hbm, out_hbm, idx_all, rows, gsem, osem):
        wid = lax.axis_index("s") * NC + lax.axis_index("c")
        base0 = wid * b_per_w

        pltpu.sync_copy(x_hbm.at[pl.ds(base0, b_per_w)], idx_all)

        def start_gather(chunk, b):
            idx_sl = idx_all.at[pl.ds(chunk * _CHUNK, _CHUNK)]
            pltpu.async_copy(table_hbm.at[idx_sl], rows[b], gsem[b])

        def wait_gather(chunk, b):
            idx_sl = idx_all.at[pl.ds(chunk * _CHUNK, _CHUNK)]
            pltpu.make_async_copy(table_hbm.at[idx_sl], rows[b], gsem[b]).wait()

        def start_store(chunk, b):
            dst = out_hbm.at[pl.ds(base0 + chunk * _CHUNK, _CHUNK)]
            pltpu.async_copy(rows[b], dst, osem[b])

        def wait_store(chunk, b):
            dst = out_hbm.at[pl.ds(base0 + chunk * _CHUNK, _CHUNK)]
            pltpu.make_async_copy(rows[b], dst, osem[b]).wait()

        for b in range(_NBUF):
            start_gather(b, b)

        def group(g, carry):
            for b in range(_NBUF):
                chunk = g * _NBUF + b
                wait_gather(chunk, b)
                start_store(chunk, b)
                wait_store(chunk, b)
                start_gather(chunk + _NBUF, b)
            return carry

        lax.fori_loop(0, n_groups - 1, group, 0)

        for b in range(_NBUF):
            chunk = (n_groups - 1) * _NBUF + b
            wait_gather(chunk, b)
            start_store(chunk, b)
            wait_store(chunk, b)

    return emb


def kernel(x, table):
    Bm, Bh = x.shape
    B = Bm * Bh
    D = table.shape[1]
    emb = _emb_call(B, D, 2, 16)
    xf = x.reshape(-1).astype(jnp.int32)
    out = emb(xf, table)
    return out.reshape(Bm, Bh, D)


# speedup vs baseline: 1.2388x; 1.1126x over previous
import functools

import jax
import jax.numpy as jnp
from jax import lax
from jax.experimental import pallas as pl
from jax.experimental.pallas import tpu as pltpu
from jax.experimental.pallas import tpu_sc as plsc

_CHUNK = 800
_NBUF = 4


def _sc_gather(B, D, NC, NS):
    NW = NC * NS
    b_per_w = B // NW
    n_chunks = b_per_w // _CHUNK
    n_groups = n_chunks // _NBUF

    mesh = plsc.VectorSubcoreMesh(core_axis_name="c", subcore_axis_name="s")

    @functools.partial(
        pl.kernel,
        out_type=jax.ShapeDtypeStruct((B, D), jnp.float32),
        mesh=mesh,
        scratch_types=[
            pltpu.VMEM((b_per_w,), jnp.int32),
            [pltpu.VMEM((_CHUNK, D), jnp.float32) for _ in range(_NBUF)],
            [pltpu.SemaphoreType.DMA for _ in range(_NBUF)],
            [pltpu.SemaphoreType.DMA for _ in range(_NBUF)],
        ],
        compiler_params=pltpu.CompilerParams(use_tc_tiling_on_sc=False),
    )
    def emb(x_hbm, table_hbm, out_hbm, idx_all, rows, gsem, osem):
        wid = lax.axis_index("s") * NC + lax.axis_index("c")
        base0 = wid * b_per_w

        pltpu.sync_copy(x_hbm.at[pl.ds(base0, b_per_w)], idx_all)

        def start_gather(chunk, b):
            idx_sl = idx_all.at[pl.ds(chunk * _CHUNK, _CHUNK)]
            pltpu.async_copy(table_hbm.at[idx_sl], rows[b], gsem[b])

        def wait_gather(chunk, b):
            idx_sl = idx_all.at[pl.ds(chunk * _CHUNK, _CHUNK)]
            pltpu.make_async_copy(table_hbm.at[idx_sl], rows[b], gsem[b]).wait()

        def start_store(chunk, b):
            dst = out_hbm.at[pl.ds(base0 + chunk * _CHUNK, _CHUNK)]
            pltpu.async_copy(rows[b], dst, osem[b])

        def wait_store(chunk, b):
            dst = out_hbm.at[pl.ds(base0 + chunk * _CHUNK, _CHUNK)]
            pltpu.make_async_copy(rows[b], dst, osem[b]).wait()

        for b in range(_NBUF):
            start_gather(b, b)

        def group(g, carry):
            for b in range(_NBUF):
                chunk = g * _NBUF + b
                wait_gather(chunk, b)
                start_store(chunk, b)
                wait_store(chunk, b)
                start_gather(chunk + _NBUF, b)
            return carry

        lax.fori_loop(0, n_groups - 1, group, 0)

        for b in range(_NBUF):
            chunk = (n_groups - 1) * _NBUF + b
            wait_gather(chunk, b)
            start_store(chunk, b)
            wait_store(chunk, b)

    return emb


def _tc_transpose_table(tbl_t):
    Dm, V = tbl_t.shape
    BLK = 4096
    grid = pl.cdiv(V, BLK)

    def body(in_ref, out_ref, scr):
        scr[:, 0:Dm] = jnp.transpose(in_ref[...], (1, 0))
        for s in range(4):
            z = scr[pl.Slice(s, BLK // 4, 4), :]
            out_ref[:, s * Dm:(s + 1) * Dm] = z[:, 0:Dm]

    return pl.pallas_call(
        body,
        grid=(grid,),
        in_specs=[pl.BlockSpec((Dm, BLK), lambda j: (0, j))],
        out_specs=pl.BlockSpec((BLK // 4, 128), lambda j: (j, 0)),
        out_shape=jax.ShapeDtypeStruct((V * Dm // 128, 128), jnp.float32),
        scratch_shapes=[pltpu.VMEM((BLK, 128), jnp.float32)],
    )(tbl_t)


def _tc_out_transpose(g204, NB, H, D):
    BB = 128
    grid = NB // BB
    rows_per_block = BB * H * D // 128

    def body(in_ref, out_ref):
        xb = in_ref[...]
        x3 = xb.reshape(BB, H, D)
        out_ref[...] = jnp.transpose(x3, (1, 2, 0))

    return pl.pallas_call(
        body,
        grid=(grid,),
        in_specs=[pl.BlockSpec((rows_per_block, 128), lambda j: (j, 0))],
        out_specs=pl.BlockSpec((H, D, BB), lambda j: (0, 0, j)),
        out_shape=jax.ShapeDtypeStruct((H, D, NB), jnp.float32),
    )(g204)


def kernel(x, table):
    NB, H = x.shape
    V, D = table.shape
    B = NB * H

    tbl_lin = _tc_transpose_table(table.T).reshape(V, D)
    xf = x.reshape(-1).astype(jnp.int32)
    g = _sc_gather(B, D, 2, 16)(xf, tbl_lin)
    return g.reshape(NB, H, D)


# speedup vs baseline: 2.7466x; 2.2172x over previous
import functools

import jax
import jax.numpy as jnp
from jax import lax
from jax.experimental import pallas as pl
from jax.experimental.pallas import tpu as pltpu
from jax.experimental.pallas import tpu_sc as plsc

_BCHUNK = 16
_NBUF = 4
_HPAD = 64


def _sc_gather(NB, H, D, NC, NS):
    B = NB * H
    NW = NC * NS
    b_per_w = B // NW
    nb_per_w = NB // NW
    chunk = _BCHUNK * H
    n_chunks = nb_per_w // _BCHUNK
    n_groups = n_chunks // _NBUF

    mesh = plsc.VectorSubcoreMesh(core_axis_name="c", subcore_axis_name="s")

    @functools.partial(
        pl.kernel,
        out_type=jax.ShapeDtypeStruct((NB, _HPAD, D), jnp.float32),
        mesh=mesh,
        scratch_types=[
            pltpu.VMEM((b_per_w,), jnp.int32),
            [pltpu.VMEM((chunk, D), jnp.float32) for _ in range(_NBUF)],
            [pltpu.SemaphoreType.DMA for _ in range(_NBUF)],
            [pltpu.SemaphoreType.DMA for _ in range(_NBUF)],
        ],
        compiler_params=pltpu.CompilerParams(use_tc_tiling_on_sc=False),
    )
    def emb(x_hbm, table_hbm, out_hbm, idx_all, rows, gsem, osem):
        wid = lax.axis_index("s") * NC + lax.axis_index("c")
        base0 = wid * b_per_w
        brow0 = wid * nb_per_w

        pltpu.sync_copy(x_hbm.at[pl.ds(base0, b_per_w)], idx_all)

        def start_gather(c, b):
            idx_sl = idx_all.at[pl.ds(c * chunk, chunk)]
            pltpu.async_copy(table_hbm.at[idx_sl], rows[b], gsem[b])

        def wait_gather(c, b):
            idx_sl = idx_all.at[pl.ds(c * chunk, chunk)]
            pltpu.make_async_copy(table_hbm.at[idx_sl], rows[b], gsem[b]).wait()

        def store_refs(c, b, bl):
            src = rows[b].at[pl.ds(bl * H, H)]
            dst = out_hbm.at[brow0 + c * _BCHUNK + bl, pl.ds(0, H)]
            return src, dst

        def start_store(c, b):
            for bl in range(_BCHUNK):
                src, dst = store_refs(c, b, bl)
                pltpu.async_copy(src, dst, osem[b])

        def wait_store(c, b):
            for bl in range(_BCHUNK):
                src, dst = store_refs(c, b, bl)
                pltpu.make_async_copy(src, dst, osem[b]).wait()

        for b in range(_NBUF):
            start_gather(b, b)

        def group(g, carry):
            for b in range(_NBUF):
                c = g * _NBUF + b
                wait_gather(c, b)
                start_store(c, b)
                wait_store(c, b)
                start_gather(c + _NBUF, b)
            return carry

        lax.fori_loop(0, n_groups - 1, group, 0)

        for b in range(_NBUF):
            c = (n_groups - 1) * _NBUF + b
            wait_gather(c, b)
            start_store(c, b)
            wait_store(c, b)

    return emb


def _tc_transpose_table(tbl_t):
    Dm, V = tbl_t.shape
    BLK = 4096
    grid = pl.cdiv(V, BLK)

    def body(in_ref, out_ref, scr):
        scr[:, 0:Dm] = jnp.transpose(in_ref[...], (1, 0))
        for s in range(4):
            z = scr[pl.Slice(s, BLK // 4, 4), :]
            out_ref[:, s * Dm:(s + 1) * Dm] = z[:, 0:Dm]

    return pl.pallas_call(
        body,
        grid=(grid,),
        in_specs=[pl.BlockSpec((Dm, BLK), lambda j: (0, j))],
        out_specs=pl.BlockSpec((BLK // 4, 128), lambda j: (j, 0)),
        out_shape=jax.ShapeDtypeStruct((V * Dm // 128, 128), jnp.float32),
        scratch_shapes=[pltpu.VMEM((BLK, 128), jnp.float32)],
    )(tbl_t)


def _tc_out_transpose(gp, NB, H, D):
    BB = 128
    grid = NB // BB
    rpb = _HPAD * D // 128
    rows_per_block = BB * rpb

    def body(in_ref, out_ref):
        inr = in_ref.reshape(BB, rpb, 128)
        for h in range(H):
            e = h * D
            yh = inr[:, e // 128, (e % 128):(e % 128) + D]
            out_ref[h] = jnp.transpose(yh, (1, 0))

    return pl.pallas_call(
        body,
        grid=(grid,),
        in_specs=[pl.BlockSpec((rows_per_block, 128), lambda j: (j, 0))],
        out_specs=pl.BlockSpec((H, D, BB), lambda j: (0, 0, j)),
        out_shape=jax.ShapeDtypeStruct((H, D, NB), jnp.float32),
    )(gp)


def kernel(x, table):
    NB, H = x.shape
    V, D = table.shape

    tbl_lin = _tc_transpose_table(table.T).reshape(V, D)
    xf = x.reshape(-1).astype(jnp.int32)
    g = _sc_gather(NB, H, D, 2, 16)(xf, tbl_lin)
    w = _tc_out_transpose(g.reshape(NB * _HPAD * D // 128, 128), NB, H, D)
    return jnp.transpose(w, (2, 0, 1))


# speedup vs baseline: 3.2951x; 1.1997x over previous
import functools

import jax
import jax.numpy as jnp
from jax import lax
from jax.experimental import pallas as pl
from jax.experimental.pallas import tpu as pltpu
from jax.experimental.pallas import tpu_sc as plsc

_BCHUNK = 16
_NBUF = 4
_HPAD = 64


def _sc_gather(NB, H, D, NC, NS):
    B = NB * H
    NW = NC * NS
    b_per_w = B // NW
    nb_per_w = NB // NW
    chunk = _BCHUNK * H
    n_chunks = nb_per_w // _BCHUNK
    n_groups = n_chunks // _NBUF

    mesh = plsc.VectorSubcoreMesh(core_axis_name="c", subcore_axis_name="s")

    @functools.partial(
        pl.kernel,
        out_type=jax.ShapeDtypeStruct((NB, _HPAD, D), jnp.float32),
        mesh=mesh,
        scratch_types=[
            pltpu.VMEM((b_per_w,), jnp.int32),
            [pltpu.VMEM((chunk, D), jnp.float32) for _ in range(_NBUF)],
            [pltpu.SemaphoreType.DMA for _ in range(_NBUF)],
            [pltpu.SemaphoreType.DMA for _ in range(_NBUF)],
        ],
        compiler_params=pltpu.CompilerParams(use_tc_tiling_on_sc=False),
    )
    def emb(x_hbm, table_hbm, out_hbm, idx_all, rows, gsem, osem):
        wid = lax.axis_index("s") * NC + lax.axis_index("c")
        base0 = wid * b_per_w
        brow0 = wid * nb_per_w

        pltpu.sync_copy(x_hbm.at[pl.ds(base0, b_per_w)], idx_all)

        def start_gather(c, b):
            idx_sl = idx_all.at[pl.ds(c * chunk, chunk)]
            pltpu.async_copy(table_hbm.at[idx_sl], rows[b], gsem[b])

        def wait_gather(c, b):
            idx_sl = idx_all.at[pl.ds(c * chunk, chunk)]
            pltpu.make_async_copy(table_hbm.at[idx_sl], rows[b], gsem[b]).wait()

        def store_refs(c, b, bl):
            src = rows[b].at[pl.ds(bl * H, H)]
            dst = out_hbm.at[brow0 + c * _BCHUNK + bl, pl.ds(0, H)]
            return src, dst

        def start_store(c, b):
            for bl in range(_BCHUNK):
                src, dst = store_refs(c, b, bl)
                pltpu.async_copy(src, dst, osem[b])

        def wait_store(c, b):
            for bl in range(_BCHUNK):
                src, dst = store_refs(c, b, bl)
                pltpu.make_async_copy(src, dst, osem[b]).wait()

        for b in range(_NBUF):
            start_gather(b, b)

        def group(g, carry):
            for b in range(_NBUF):
                c = g * _NBUF + b
                wait_gather(c, b)
                start_store(c, b)
                wait_store(c, b)
                start_gather(c + _NBUF, b)
            return carry

        lax.fori_loop(0, n_groups - 1, group, 0)

        for b in range(_NBUF):
            c = (n_groups - 1) * _NBUF + b
            wait_gather(c, b)
            start_store(c, b)
            wait_store(c, b)

    return emb


def _tc_transpose_table(tbl_t):
    Dm, V = tbl_t.shape
    BLK = 4096
    grid = pl.cdiv(V, BLK)

    NSP = 4
    SUB = BLK // NSP

    def body(in_ref, out_ref, scr):
        for t in range(NSP):
            scr[t * SUB:(t + 1) * SUB, 0:Dm] = jnp.transpose(
                in_ref[:, t * SUB:(t + 1) * SUB], (1, 0)
            )
        for t in range(NSP):
            q0 = t * (SUB // 4)
            for s in range(4):
                z = scr[pl.Slice(t * SUB + s, SUB // 4, 4), :]
                out_ref[q0:q0 + SUB // 4, s * Dm:(s + 1) * Dm] = z[:, 0:Dm]

    return pl.pallas_call(
        body,
        grid=(grid,),
        in_specs=[pl.BlockSpec((Dm, BLK), lambda j: (0, j))],
        out_specs=pl.BlockSpec((BLK // 4, 128), lambda j: (j, 0)),
        out_shape=jax.ShapeDtypeStruct((V * Dm // 128, 128), jnp.float32),
        scratch_shapes=[pltpu.VMEM((BLK, 128), jnp.float32)],
    )(tbl_t)


def _tc_out_transpose(gp, NB, H, D):
    BB = 128
    grid = NB // BB
    rpb = _HPAD * D // 128
    rows_per_block = BB * rpb

    def body(in_ref, out_ref):
        inr = in_ref.reshape(BB, rpb, 128)
        hpr = 128 // D
        for r in range(pl.cdiv(H, hpr)):
            t = jnp.transpose(inr[:, r, :], (1, 0))
            for q in range(hpr):
                h = hpr * r + q
                if h < H:
                    out_ref[h] = t[q * D:(q + 1) * D, :]

    return pl.pallas_call(
        body,
        grid=(grid,),
        in_specs=[pl.BlockSpec((rows_per_block, 128), lambda j: (j, 0))],
        out_specs=pl.BlockSpec((H, D, BB), lambda j: (0, 0, j)),
        out_shape=jax.ShapeDtypeStruct((H, D, NB), jnp.float32),
    )(gp)


def kernel(x, table):
    NB, H = x.shape
    V, D = table.shape

    tbl_lin = _tc_transpose_table(table.T).reshape(V, D)
    xf = x.reshape(-1).astype(jnp.int32)
    g = _sc_gather(NB, H, D, 2, 16)(xf, tbl_lin)
    w = _tc_out_transpose(g.reshape(NB * _HPAD * D // 128, 128), NB, H, D)
    return jnp.transpose(w, (2, 0, 1))
